# Initial kernel scaffold; baseline (speedup 1.0000x reference)
import jax
import jax.numpy as jnp
from jax.experimental import pallas as pl


def kernel(node_features, edge_index, W1, b1, W2, b2, W3, b3, W4, b4):
    raise NotImplementedError("write your pallas kernel here")


# speedup vs baseline: 9.2339x; 9.2339x over previous
import functools

import jax
import jax.numpy as jnp
from jax import lax
from jax.experimental import pallas as pl
from jax.experimental.pallas import tpu as pltpu
from jax.experimental.pallas import tpu_sc as plsc

_NC = 2
_NS = 16
_NW = _NC * _NS
_C = 80


def _mesh():
    return plsc.VectorSubcoreMesh(
        core_axis_name="c", subcore_axis_name="s",
        num_cores=_NC, num_subcores=_NS)


def _acc_rows(n):
    per_tile = -(-n // _NS)
    rpt = -(-per_tile // _C) * _C
    return rpt, rpt * _NS


def _deg_partials(dst, n, e):
    rpt, np_ = _acc_rows(n)
    ep = e // _NW
    nch = ep // _C

    @functools.partial(
        pl.kernel,
        out_type=jax.ShapeDtypeStruct((_NC * np_, 16), jnp.float32),
        mesh=_mesh(),
        scratch_types=[
            pltpu.VMEM((_C,), jnp.int32),
            pltpu.VMEM((_C, 16), jnp.float32),
            pltpu.VMEM_SHARED((np_, 16), jnp.float32),
        ],
    )
    def deg_k(dst_hbm, out_hbm, dst_v, buf_v, acc):
        c = lax.axis_index("c")
        s = lax.axis_index("s")
        wid = s * _NC + c
        row0 = s * rpt

        def fill(val16):
            def fb(r, _):
                buf_v[r, :] = val16
                return 0
            lax.fori_loop(0, _C, fb, 0)

        fill(jnp.zeros((16,), jnp.float32))

        def zout(j, _):
            pltpu.sync_copy(buf_v, acc.at[pl.ds(row0 + j * _C, _C)])
            return 0
        lax.fori_loop(0, rpt // _C, zout, 0)

        fill(jnp.ones((16,), jnp.float32))
        plsc.subcore_barrier()

        def body(i, _):
            base = wid * ep + i * _C
            pltpu.sync_copy(dst_hbm.at[pl.ds(base, _C)], dst_v)
            pltpu.sync_copy(buf_v, acc.at[dst_v], add=True)
            return 0
        lax.fori_loop(0, nch, body, 0)

        plsc.subcore_barrier()

        def cout(j, _):
            pltpu.sync_copy(acc.at[pl.ds(row0 + j * _C, _C)], buf_v)
            pltpu.sync_copy(buf_v, out_hbm.at[pl.ds(c * np_ + row0 + j * _C, _C)])
            return 0
        lax.fori_loop(0, rpt // _C, cout, 0)

    return deg_k(dst).reshape(_NC, np_, 16)


def _aggregate(src, dst, hs, n, e, d):
    rpt, np_ = _acc_rows(n)
    ep = e // _NW
    nch = ep // _C

    @functools.partial(
        pl.kernel,
        out_type=jax.ShapeDtypeStruct((_NC * np_, d), jnp.float32),
        mesh=_mesh(),
        scratch_types=[
            pltpu.VMEM((_C,), jnp.int32),
            pltpu.VMEM((_C,), jnp.int32),
            pltpu.VMEM((_C, d), jnp.float32),
            pltpu.VMEM_SHARED((np_, d), jnp.float32),
            pltpu.SemaphoreType.DMA,
        ],
    )
    def agg_k(src_hbm, dst_hbm, hs_hbm, out_hbm, src_v, dst_v, rows_v, acc, sem):
        c = lax.axis_index("c")
        s = lax.axis_index("s")
        wid = s * _NC + c
        row0 = s * rpt
        z16 = jnp.zeros((16,), jnp.float32)

        def zrow(r, _):
            def zcol(k, _):
                rows_v[r, pl.ds(k * 16, 16)] = z16
                return 0
            lax.fori_loop(0, d // 16, zcol, 0)
            return 0
        lax.fori_loop(0, _C, zrow, 0)

        def zout(j, _):
            pltpu.sync_copy(rows_v, acc.at[pl.ds(row0 + j * _C, _C)])
            return 0
        lax.fori_loop(0, rpt // _C, zout, 0)

        plsc.subcore_barrier()

        def body(i, _):
            base = wid * ep + i * _C
            pltpu.sync_copy(src_hbm.at[pl.ds(base, _C)], src_v)
            pltpu.sync_copy(dst_hbm.at[pl.ds(base, _C)], dst_v)
            pltpu.async_copy(hs_hbm.at[src_v], rows_v, sem).wait()
            pltpu.sync_copy(rows_v, acc.at[dst_v], add=True)
            return 0
        lax.fori_loop(0, nch, body, 0)

        plsc.subcore_barrier()

        def cout(j, _):
            pltpu.sync_copy(acc.at[pl.ds(row0 + j * _C, _C)], rows_v)
            pltpu.sync_copy(rows_v, out_hbm.at[pl.ds(c * np_ + row0 + j * _C, _C)])
            return 0
        lax.fori_loop(0, rpt // _C, cout, 0)

    return agg_k(src, dst, hs).reshape(_NC, np_, d)


_R = 2000


def _tc_first(degp, x, w1):
    n, din = x.shape
    dh = w1.shape[1]

    def body(deg_ref, x_ref, w_ref, dis_ref, hs_ref):
        deg = deg_ref[0, :, 0:1] + deg_ref[1, :, 0:1] + 1.0
        dis = lax.rsqrt(deg)
        dis_ref[...] = dis
        hs_ref[...] = dis * jnp.dot(x_ref[...], w_ref[...],
                                    preferred_element_type=jnp.float32)

    return pl.pallas_call(
        body,
        grid=(n // _R,),
        in_specs=[
            pl.BlockSpec((_NC, _R, 16), lambda i: (0, i, 0)),
            pl.BlockSpec((_R, din), lambda i: (i, 0)),
            pl.BlockSpec((din, dh), lambda i: (0, 0)),
        ],
        out_specs=[
            pl.BlockSpec((_R, 1), lambda i: (i, 0)),
            pl.BlockSpec((_R, dh), lambda i: (i, 0)),
        ],
        out_shape=[
            jax.ShapeDtypeStruct((n, 1), jnp.float32),
            jax.ShapeDtypeStruct((n, dh), jnp.float32),
        ],
    )(degp, x, w1)


def _tc_mid(p, hs, dis, b, w):
    n, d = hs.shape
    dn = w.shape[1]

    def body(p_ref, hs_ref, dis_ref, b_ref, w_ref, out_ref):
        a = p_ref[0] + p_ref[1] + hs_ref[...]
        h = jnp.maximum(dis_ref[...] * a + b_ref[...], 0.0)
        out_ref[...] = dis_ref[...] * jnp.dot(h, w_ref[...],
                                              preferred_element_type=jnp.float32)

    return pl.pallas_call(
        body,
        grid=(n // _R,),
        in_specs=[
            pl.BlockSpec((_NC, _R, d), lambda i: (0, i, 0)),
            pl.BlockSpec((_R, d), lambda i: (i, 0)),
            pl.BlockSpec((_R, 1), lambda i: (i, 0)),
            pl.BlockSpec((1, d), lambda i: (0, 0)),
            pl.BlockSpec((d, dn), lambda i: (0, 0)),
        ],
        out_specs=pl.BlockSpec((_R, dn), lambda i: (i, 0)),
        out_shape=jax.ShapeDtypeStruct((n, dn), jnp.float32),
    )(p, hs, dis, b, w)


def _tc_last(p, hs, dis, b3, w4, b4):
    n, d = hs.shape
    dn = w4.shape[1]

    dv = w4.shape[0]

    def body(p_ref, hs_ref, dis_ref, b3_ref, w_ref, b4_ref, out_ref):
        a = p_ref[0] + p_ref[1] + hs_ref[...]
        h = (dis_ref[...] * a)[:, :dv] + b3_ref[...]
        out_ref[...] = jax.nn.sigmoid(
            jnp.dot(h, w_ref[...], preferred_element_type=jnp.float32)
            + b4_ref[...])

    return pl.pallas_call(
        body,
        grid=(n // _R,),
        in_specs=[
            pl.BlockSpec((_NC, _R, d), lambda i: (0, i, 0)),
            pl.BlockSpec((_R, d), lambda i: (i, 0)),
            pl.BlockSpec((_R, 1), lambda i: (i, 0)),
            pl.BlockSpec((1, dv), lambda i: (0, 0)),
            pl.BlockSpec((dv, dn), lambda i: (0, 0)),
            pl.BlockSpec((1, dn), lambda i: (0, 0)),
        ],
        out_specs=pl.BlockSpec((_R, dn), lambda i: (i, 0)),
        out_shape=jax.ShapeDtypeStruct((n, dn), jnp.float32),
    )(p, hs, dis, b3, w4, b4)


def kernel(node_features, edge_index, W1, b1, W2, b2, W3, b3, W4, b4):
    x = node_features
    n, _ = x.shape
    e = edge_index.shape[1]
    src = edge_index[0]
    dst = edge_index[1]
    dh = W1.shape[1]
    do = W3.shape[1]

    degp = _deg_partials(dst, n, e)
    dis, hs = _tc_first(degp, x, W1)

    p = _aggregate(src, dst, hs, n, e, dh)
    hs = _tc_mid(p, hs, dis, b1.reshape(1, -1), W2)
    for _ in range(3):
        p = _aggregate(src, dst, hs, n, e, dh)
        hs = _tc_mid(p, hs, dis, b2.reshape(1, -1), W2)
    w3p = jnp.pad(W3, ((0, 0), (0, dh - do)))
    p = _aggregate(src, dst, hs, n, e, dh)
    hs = _tc_mid(p, hs, dis, b2.reshape(1, -1), w3p)

    p = _aggregate(src, dst, hs, n, e, dh)
    return _tc_last(p, hs, dis, b3.reshape(1, -1), W4, b4.reshape(1, -1))
